# Initial kernel scaffold; baseline (speedup 1.0000x reference)
import jax
import jax.numpy as jnp
from jax.experimental import pallas as pl


def kernel(y_pred, distance_maps):
    raise NotImplementedError("write your pallas kernel here")


# speedup vs baseline: 1.3595x; 1.3595x over previous
import functools

import jax
import jax.numpy as jnp
from jax import lax
from jax.experimental import pallas as pl
from jax.experimental.pallas import tpu as pltpu
from jax.experimental.pallas import tpu_sc as plsc

_W = 256
_M = _W * _W
_L = 16
_NC, _NS = 2, 16
_NW = _NC * _NS


@functools.lru_cache(maxsize=None)
def _build_sc_call(B, P):
    assert B % _NW == 0
    assert P % _L == 0
    spw = B // _NW
    chunks = P // _L

    mesh = plsc.VectorSubcoreMesh(core_axis_name="c", subcore_axis_name="s")

    @functools.partial(
        pl.kernel,
        out_type=jax.ShapeDtypeStruct((_NW, _L), jnp.float32),
        mesh=mesh,
        compiler_params=pltpu.CompilerParams(needs_layout_passes=False),
        scratch_types=[
            pltpu.VMEM((_M,), jnp.float32),
            pltpu.VMEM((2 * P,), jnp.float32),
            pltpu.VMEM((_L,), jnp.float32),
        ],
    )
    def sc_call(y_hbm, d_hbm, out_hbm, dv, yv, acc_v):
        wid = lax.axis_index("s") * _NC + lax.axis_index("c")
        lanes = lax.iota(jnp.int32, _L)

        def chunk_body(i, acc):
            base = i * (2 * _L)
            xidx = base + 2 * lanes
            x = plsc.load_gather(yv, [xidx])
            y = plsc.load_gather(yv, [xidx + 1])
            xi = (x * 256.0).astype(jnp.int32)
            yi = (y * 256.0).astype(jnp.int32)
            flat = jnp.clip(xi + yi * _W, 0, _M - 1)
            g = plsc.load_gather(dv, [flat])
            t = g * 512.0 - 254.0
            return acc + t * t

        total = jnp.zeros((_L,), jnp.float32)
        for s in range(spw):
            sample = wid * spw + s
            pltpu.sync_copy(d_hbm.at[sample], dv)
            pltpu.sync_copy(y_hbm.at[sample], yv)
            total = lax.fori_loop(0, chunks, chunk_body, total)

        acc_v[...] = total
        pltpu.sync_copy(acc_v, out_hbm.at[wid])

    return sc_call


def kernel(y_pred, distance_maps):
    B = y_pred.shape[0]
    P = y_pred.shape[1] * y_pred.shape[2] // 2
    yf = y_pred.reshape(B, 2 * P)
    df = distance_maps.reshape(B, _M)
    partial = _build_sc_call(B, P)(yf, df)
    return jnp.sum(partial) * (1.0 / (B * P))
